# baseline (speedup 1.0000x reference)
import jax
import jax.numpy as jnp
import numpy as np
from jax import lax
from jax.experimental import pallas as pl
from jax.experimental.pallas import tpu as pltpu
from jax.experimental.pallas import tpu_sc as plsc

V = 642
F = 1280
H = 1001
W = 1001
N = 1048576

NC = 2
NS = 16
NW = NC * NS
L = 16
PT = N // NW
C = 4096
NG = C // L
NCH = PT // C
JROWS = C // 128

_MAGIC = np.float32(12582912.0)
_PI = np.float32(3.14159265358979)


def _sinp(x, terms):
    t = x * x
    r = np.float32(terms[0])
    for c in terms[1:]:
        r = r * t + np.float32(c)
    return x * r


def _cosp(x, terms):
    t = x * x
    r = np.float32(terms[0])
    for c in terms[1:]:
        r = r * t + np.float32(c)
    return r


_SIN_HI = (-1.0 / 39916800, 1.0 / 362880, -1.0 / 5040, 1.0 / 120, -1.0 / 6, 1.0)
_COS_HI = (1.0 / 479001600, -1.0 / 3628800, 1.0 / 40320, -1.0 / 720, 1.0 / 24,
           -0.5, 1.0)
_SIN_LO = (1.0 / 362880, -1.0 / 5040, 1.0 / 120, -1.0 / 6, 1.0)
_COS_LO = (-1.0 / 3628800, 1.0 / 40320, -1.0 / 720, 1.0 / 24, -0.5, 1.0)


def _sphere(u, v, sin_t, cos_t):
    h = _PI * (u - np.float32(0.5))
    th = _PI * (v - np.float32(0.5))
    sh = _sinp(h, sin_t)
    ch = _cosp(h, cos_t)
    sphi = np.float32(2.0) * sh * ch
    cphi = np.float32(1.0) - np.float32(2.0) * sh * sh
    st = _sinp(th, sin_t)
    ct = _cosp(th, cos_t)
    return ct * cphi, ct * sphi, st


def _rsqrt(s, iters):
    i = plsc.bitcast(s, jnp.int32)
    i = np.int32(0x5F3759DF) - lax.shift_right_logical(i, 1)
    y = plsc.bitcast(i, jnp.float32)
    for _ in range(iters):
        y = y * (np.float32(1.5) - np.float32(0.5) * s * y * y)
    return y


def _body(uvt, verts, uvv, faces, finds, out_f,
          s_faces, s_uvv, s_verts, s_struct, s_u, s_v, s_px, s_py, s_pz,
          s_lin, s_fi, s_out, sem_g):
    cid = lax.axis_index("c")
    sid = lax.axis_index("s")
    wid = sid * NC + cid

    pltpu.sync_copy(faces, s_faces)
    pltpu.sync_copy(uvv, s_uvv)
    pltpu.sync_copy(verts, s_verts)

    iota = lax.iota(jnp.int32, L)

    def build(g):
        fvec3 = (g * L + iota) * 3
        vi = [plsc.load_gather(s_faces, [fvec3 + k]) for k in range(3)]
        vi2 = [v2 * 2 for v2 in vi]
        uu = [plsc.load_gather(s_uvv, [vi2[k]]) for k in range(3)]
        vv = [plsc.load_gather(s_uvv, [vi2[k] + 1]) for k in range(3)]
        A = _sphere(uu[0], vv[0], _SIN_HI, _COS_HI)
        B = _sphere(uu[1], vv[1], _SIN_HI, _COS_HI)
        Cc = _sphere(uu[2], vv[2], _SIN_HI, _COS_HI)
        AB = [B[d] - A[d] for d in range(3)]
        AC = [Cc[d] - A[d] for d in range(3)]
        crx = AB[1] * AC[2] - AB[2] * AC[1]
        cry = AB[2] * AC[0] - AB[0] * AC[2]
        crz = AB[0] * AC[1] - AB[1] * AC[0]
        s0 = crx * crx + cry * cry + crz * crz
        ia = _rsqrt(s0, 3)
        gl = g * L
        for d in range(3):
            s_struct[pl.ds((0 + d) * F + gl, L)] = A[d]
            s_struct[pl.ds((3 + d) * F + gl, L)] = AB[d]
            s_struct[pl.ds((6 + d) * F + gl, L)] = AC[d]
        for k in range(3):
            vi3 = vi[k] * 3
            for d in range(3):
                s_struct[pl.ds((9 + k * 3 + d) * F + gl, L)] = plsc.load_gather(
                    s_verts, [vi3 + d])
        s_struct[pl.ds(18 * F + gl, L)] = ia

    plsc.parallel_loop(0, F // L, unroll=2)(build)

    def lin_pass(par):
        poff = par * C

        def linb(g):
            gsl0 = pl.ds(poff + g * L, L)
            u = s_u[gsl0]
            v = s_v[gsl0]
            x = (u * np.float32(1000.0) + _MAGIC) - _MAGIC
            y = (v * np.float32(1000.0) + _MAGIC) - _MAGIC
            lin = y.astype(jnp.int32) * W + x.astype(jnp.int32)
            s_lin[par * JROWS + g // 8, pl.ds((g % 8) * L, L)] = lin
            px, py, pz = _sphere(u, v, _SIN_LO, _COS_LO)
            gsl = pl.ds(poff + g * L, L)
            s_px[gsl] = px
            s_py[gsl] = py
            s_pz[gsl] = pz

        plsc.parallel_loop(0, NG, unroll=8)(linb)

    def math_pass(opar, obase):
        pltpu.make_async_copy(finds.at[pl.ds(0, C)],
                              s_fi.at[pl.ds(opar * C, C)], sem_g).wait()
        poff = opar * C

        def mathb(g):
            gsl = pl.ds(poff + g * L, L)
            fi = s_fi[gsl]
            px = s_px[gsl]
            py = s_py[gsl]
            pz = s_pz[gsl]

            def ld(r):
                return plsc.load_gather(s_struct, [fi + r * F])

            ax, ay, az = ld(0), ld(1), ld(2)
            abx, aby, abz = ld(3), ld(4), ld(5)
            acx, acy, acz = ld(6), ld(7), ld(8)
            bcx, bcy, bcz = acx - abx, acy - aby, acz - abz
            apx, apy, apz = px - ax, py - ay, pz - az
            bpx, bpy, bpz = apx - abx, apy - aby, apz - abz
            c1x = aby * apz - abz * apy
            c1y = abz * apx - abx * apz
            c1z = abx * apy - aby * apx
            c2x = acy * apz - acz * apy
            c2y = acz * apx - acx * apz
            c2z = acx * apy - acy * apx
            c3x = bcy * bpz - bcz * bpy
            c3y = bcz * bpx - bcx * bpz
            c3z = bcx * bpy - bcy * bpx
            s1 = c1x * c1x + c1y * c1y + c1z * c1z
            s2 = c2x * c2x + c2y * c2y + c2z * c2z
            s3 = c3x * c3x + c3y * c3y + c3z * c3z
            ia = ld(18)
            ww = s1 * _rsqrt(s1, 1) * ia
            vv_ = s2 * _rsqrt(s2, 1) * ia
            uu_ = s3 * _rsqrt(s3, 1) * ia
            l1 = jnp.maximum(uu_ + vv_ + ww, np.float32(1e-12))
            r = np.float32(1.0) / l1
            ru, rv, rw = uu_ * r, vv_ * r, ww * r
            for d in range(3):
                val = ld(9 + d) * ru + ld(12 + d) * rv + ld(15 + d) * rw
                s_out[pl.ds(d * C + g * L, L)] = val

        plsc.parallel_loop(0, NG, unroll=12)(mathb)
        ob = pl.multiple_of(obase, 2048)
        for d in range(3):
            pltpu.sync_copy(s_out.at[pl.ds(d * C, C)],
                            out_f.at[pl.ds(d * N + ob, C)])

    def chunk(i, carry):
        par = lax.rem(i, 2)
        base = wid * PT + i * C
        b = pl.multiple_of(base, 2048)
        pltpu.sync_copy(uvt.at[pl.ds(b, C)], s_u.at[pl.ds(par * C, C)])
        pltpu.sync_copy(uvt.at[pl.ds(N + b, C)], s_v.at[pl.ds(par * C, C)])
        lin_pass(par)
        for j in range(JROWS):
            pltpu.async_copy(finds.at[s_lin.at[par * JROWS + j]],
                             s_fi.at[pl.ds(par * C + j * 128, 128)], sem_g)

        @pl.when(i > 0)
        def _():
            math_pass(1 - par, base - C)

        return carry

    lax.fori_loop(0, NCH, chunk, None)
    math_pass((NCH - 1) % 2, wid * PT + (NCH - 1) * C)


def kernel(uv, verts_3d, uv_verts, faces, face_inds):
    uvt = uv.T.reshape(2 * N)
    finds = jnp.bitwise_and(face_inds, np.int32(0x7FFFFFFF)).reshape(-1)
    verts_f = verts_3d.reshape(-1)
    uvv_f = uv_verts.reshape(-1)
    faces_f = faces.reshape(-1)
    mesh = plsc.VectorSubcoreMesh(core_axis_name="c", subcore_axis_name="s")
    run = pl.kernel(
        _body,
        out_type=jax.ShapeDtypeStruct((3 * N,), jnp.float32),
        mesh=mesh,
        compiler_params=pltpu.CompilerParams(needs_layout_passes=False),
        scratch_types=[
            pltpu.VMEM((F * 3,), jnp.int32),
            pltpu.VMEM((V * 2,), jnp.float32),
            pltpu.VMEM((V * 3,), jnp.float32),
            pltpu.VMEM((19 * F,), jnp.float32),
            pltpu.VMEM((2 * C,), jnp.float32),
            pltpu.VMEM((2 * C,), jnp.float32),
            pltpu.VMEM((2 * C,), jnp.float32),
            pltpu.VMEM((2 * C,), jnp.float32),
            pltpu.VMEM((2 * C,), jnp.float32),
            pltpu.VMEM((2 * JROWS, 128), jnp.int32),
            pltpu.VMEM((2 * C,), jnp.int32),
            pltpu.VMEM((3 * C,), jnp.float32),
            pltpu.SemaphoreType.DMA,
        ],
    )
    out = run(uvt, verts_f, uvv_f, faces_f, finds)
    return jnp.stack([out[:N], out[N:2 * N], out[2 * N:]], axis=1)


# speedup vs baseline: 1.0182x; 1.0182x over previous
import jax
import jax.numpy as jnp
import numpy as np
from jax import lax
from jax.experimental import pallas as pl
from jax.experimental.pallas import tpu as pltpu
from jax.experimental.pallas import tpu_sc as plsc

V = 642
F = 1280
H = 1001
W = 1001
N = 1048576

NC = 2
NS = 16
NW = NC * NS
L = 16
PT = N // NW
C = 4096
NG = C // L
NCH = PT // C
JROWS = C // 128

_MAGIC = np.float32(12582912.0)
_PI = np.float32(3.14159265358979)


def _sinp(x, terms):
    t = x * x
    r = np.float32(terms[0])
    for c in terms[1:]:
        r = r * t + np.float32(c)
    return x * r


def _cosp(x, terms):
    t = x * x
    r = np.float32(terms[0])
    for c in terms[1:]:
        r = r * t + np.float32(c)
    return r


_SIN_HI = (-1.0 / 39916800, 1.0 / 362880, -1.0 / 5040, 1.0 / 120, -1.0 / 6, 1.0)
_COS_HI = (1.0 / 479001600, -1.0 / 3628800, 1.0 / 40320, -1.0 / 720, 1.0 / 24,
           -0.5, 1.0)
_SIN_LO = (1.0 / 362880, -1.0 / 5040, 1.0 / 120, -1.0 / 6, 1.0)
_COS_LO = (-1.0 / 3628800, 1.0 / 40320, -1.0 / 720, 1.0 / 24, -0.5, 1.0)


def _sphere(u, v, sin_t, cos_t):
    h = _PI * (u - np.float32(0.5))
    th = _PI * (v - np.float32(0.5))
    sh = _sinp(h, sin_t)
    ch = _cosp(h, cos_t)
    sphi = np.float32(2.0) * sh * ch
    cphi = np.float32(1.0) - np.float32(2.0) * sh * sh
    st = _sinp(th, sin_t)
    ct = _cosp(th, cos_t)
    return ct * cphi, ct * sphi, st


def _rsqrt(s, iters):
    i = plsc.bitcast(s, jnp.int32)
    i = np.int32(0x5F3759DF) - lax.shift_right_logical(i, 1)
    y = plsc.bitcast(i, jnp.float32)
    for _ in range(iters):
        y = y * (np.float32(1.5) - np.float32(0.5) * s * y * y)
    return y


def _body(uvt, verts, uvv, faces, finds, out_f,
          s_faces, s_uvv, s_verts, s_struct, s_u, s_v, s_px, s_py, s_pz,
          s_lin, s_fi, s_out, sem_g):
    cid = lax.axis_index("c")
    sid = lax.axis_index("s")
    wid = sid * NC + cid

    pltpu.sync_copy(faces, s_faces)
    pltpu.sync_copy(uvv, s_uvv)
    pltpu.sync_copy(verts, s_verts)

    iota = lax.iota(jnp.int32, L)

    def build(g):
        fvec3 = (g * L + iota) * 3
        vi = [plsc.load_gather(s_faces, [fvec3 + k]) for k in range(3)]
        vi2 = [v2 * 2 for v2 in vi]
        uu = [plsc.load_gather(s_uvv, [vi2[k]]) for k in range(3)]
        vv = [plsc.load_gather(s_uvv, [vi2[k] + 1]) for k in range(3)]
        A = _sphere(uu[0], vv[0], _SIN_HI, _COS_HI)
        B = _sphere(uu[1], vv[1], _SIN_HI, _COS_HI)
        Cc = _sphere(uu[2], vv[2], _SIN_HI, _COS_HI)
        AB = [B[d] - A[d] for d in range(3)]
        AC = [Cc[d] - A[d] for d in range(3)]
        crx = AB[1] * AC[2] - AB[2] * AC[1]
        cry = AB[2] * AC[0] - AB[0] * AC[2]
        crz = AB[0] * AC[1] - AB[1] * AC[0]
        s0 = crx * crx + cry * cry + crz * crz
        ia = _rsqrt(s0, 3)
        gl = g * L
        for d in range(3):
            s_struct[pl.ds((0 + d) * F + gl, L)] = A[d]
            s_struct[pl.ds((3 + d) * F + gl, L)] = AB[d]
            s_struct[pl.ds((6 + d) * F + gl, L)] = AC[d]
        for k in range(3):
            vi3 = vi[k] * 3
            for d in range(3):
                s_struct[pl.ds((9 + k * 3 + d) * F + gl, L)] = plsc.load_gather(
                    s_verts, [vi3 + d])
        s_struct[pl.ds(18 * F + gl, L)] = ia

    plsc.parallel_loop(0, F // L, unroll=2)(build)

    def lin_pass(par):
        poff = par * C

        def linb(g):
            gsl0 = pl.ds(poff + g * L, L)
            u = s_u[gsl0]
            v = s_v[gsl0]
            x = (u * np.float32(1000.0) + _MAGIC) - _MAGIC
            y = (v * np.float32(1000.0) + _MAGIC) - _MAGIC
            lin = y.astype(jnp.int32) * W + x.astype(jnp.int32)
            s_lin[par * JROWS + g // 8, pl.ds((g % 8) * L, L)] = lin
            px, py, pz = _sphere(u, v, _SIN_LO, _COS_LO)
            gsl = pl.ds(poff + g * L, L)
            s_px[gsl] = px
            s_py[gsl] = py
            s_pz[gsl] = pz

        plsc.parallel_loop(0, NG, unroll=8)(linb)

    def math_pass(opar, obase):
        pltpu.make_async_copy(finds.at[pl.ds(0, C)],
                              s_fi.at[pl.ds(opar * C, C)], sem_g).wait()
        poff = opar * C

        def mathb(g):
            gsl = pl.ds(poff + g * L, L)
            fi = s_fi[gsl]
            px = s_px[gsl]
            py = s_py[gsl]
            pz = s_pz[gsl]

            def ld(r):
                return plsc.load_gather(s_struct, [fi + r * F])

            ax, ay, az = ld(0), ld(1), ld(2)
            abx, aby, abz = ld(3), ld(4), ld(5)
            acx, acy, acz = ld(6), ld(7), ld(8)
            bcx, bcy, bcz = acx - abx, acy - aby, acz - abz
            apx, apy, apz = px - ax, py - ay, pz - az
            bpx, bpy, bpz = apx - abx, apy - aby, apz - abz
            c1x = aby * apz - abz * apy
            c1y = abz * apx - abx * apz
            c1z = abx * apy - aby * apx
            c2x = acy * apz - acz * apy
            c2y = acz * apx - acx * apz
            c2z = acx * apy - acy * apx
            c3x = bcy * bpz - bcz * bpy
            c3y = bcz * bpx - bcx * bpz
            c3z = bcx * bpy - bcy * bpx
            s1 = c1x * c1x + c1y * c1y + c1z * c1z
            s2 = c2x * c2x + c2y * c2y + c2z * c2z
            s3 = c3x * c3x + c3y * c3y + c3z * c3z
            ia = ld(18)
            ww = s1 * _rsqrt(s1, 1) * ia
            vv_ = s2 * _rsqrt(s2, 1) * ia
            uu_ = s3 * _rsqrt(s3, 1) * ia
            l1 = jnp.maximum(uu_ + vv_ + ww, np.float32(1e-12))
            r = np.float32(1.0) / l1
            ru, rv, rw = uu_ * r, vv_ * r, ww * r
            for d in range(3):
                val = ld(9 + d) * ru + ld(12 + d) * rv + ld(15 + d) * rw
                s_out[pl.ds(d * C + g * L, L)] = val

        plsc.parallel_loop(0, NG, unroll=8)(mathb)
        ob = pl.multiple_of(obase, 2048)
        for d in range(3):
            pltpu.sync_copy(s_out.at[pl.ds(d * C, C)],
                            out_f.at[pl.ds(d * N + ob, C)])

    def chunk(i, carry):
        par = lax.rem(i, 2)
        base = wid * PT + i * C
        b = pl.multiple_of(base, 2048)
        pltpu.sync_copy(uvt.at[pl.ds(b, C)], s_u.at[pl.ds(par * C, C)])
        pltpu.sync_copy(uvt.at[pl.ds(N + b, C)], s_v.at[pl.ds(par * C, C)])
        lin_pass(par)
        for j in range(JROWS):
            pltpu.async_copy(finds.at[s_lin.at[par * JROWS + j]],
                             s_fi.at[pl.ds(par * C + j * 128, 128)], sem_g)

        @pl.when(i > 0)
        def _():
            math_pass(1 - par, base - C)

        return carry

    lax.fori_loop(0, NCH, chunk, None)
    math_pass((NCH - 1) % 2, wid * PT + (NCH - 1) * C)


def kernel(uv, verts_3d, uv_verts, faces, face_inds):
    uvt = uv.T.reshape(2 * N)
    finds = jnp.bitwise_and(face_inds, np.int32(0x7FFFFFFF)).reshape(-1)
    verts_f = verts_3d.reshape(-1)
    uvv_f = uv_verts.reshape(-1)
    faces_f = faces.reshape(-1)
    mesh = plsc.VectorSubcoreMesh(core_axis_name="c", subcore_axis_name="s")
    run = pl.kernel(
        _body,
        out_type=jax.ShapeDtypeStruct((3 * N,), jnp.float32),
        mesh=mesh,
        compiler_params=pltpu.CompilerParams(needs_layout_passes=False),
        scratch_types=[
            pltpu.VMEM((F * 3,), jnp.int32),
            pltpu.VMEM((V * 2,), jnp.float32),
            pltpu.VMEM((V * 3,), jnp.float32),
            pltpu.VMEM((19 * F,), jnp.float32),
            pltpu.VMEM((2 * C,), jnp.float32),
            pltpu.VMEM((2 * C,), jnp.float32),
            pltpu.VMEM((2 * C,), jnp.float32),
            pltpu.VMEM((2 * C,), jnp.float32),
            pltpu.VMEM((2 * C,), jnp.float32),
            pltpu.VMEM((2 * JROWS, 128), jnp.int32),
            pltpu.VMEM((2 * C,), jnp.int32),
            pltpu.VMEM((3 * C,), jnp.float32),
            pltpu.SemaphoreType.DMA,
        ],
    )
    out = run(uvt, verts_f, uvv_f, faces_f, finds)
    return jnp.stack([out[:N], out[N:2 * N], out[2 * N:]], axis=1)
